# Initial kernel scaffold; baseline (speedup 1.0000x reference)
import jax
import jax.numpy as jnp
from jax.experimental import pallas as pl


def kernel(x0, edge_index, W1l, b1l, W1r, W2l, b2l, W2r, W3l, b3l, W3r, Wlin, blin):
    raise NotImplementedError("write your pallas kernel here")


# speedup vs baseline: 2.2978x; 2.2978x over previous
import functools

import jax
import jax.numpy as jnp
from jax import lax
from jax.experimental import pallas as pl
from jax.experimental.pallas import tpu as pltpu
from jax.experimental.pallas import tpu_sc as plsc

N_NODES = 10000
D = 128
Wout = 64
E = 320000
NC = 2
NS = 16
NW = NC * NS
E_PAD = 327680
E_PER_TILE = E_PAD // NW
CHUNK = 128
NCHUNK = E_PER_TILE // CHUNK
ACC_N = 10240
DEGW = 16

_MESH = plsc.VectorSubcoreMesh(core_axis_name="c", subcore_axis_name="s")


def _sc_feat_body(x_hbm, src_hbm, dst_hbm, zeros_hbm, out_hbm,
                  src_v, dst_v, rows_v, acc, sem):
    c = lax.axis_index("c")
    s = lax.axis_index("s")
    wid = s * NC + c

    @pl.when(s == 0)
    def _init():
        pltpu.sync_copy(zeros_hbm, acc)

    pltpu.sync_copy(src_hbm.at[wid], src_v)
    pltpu.sync_copy(dst_hbm.at[wid], dst_v)
    plsc.subcore_barrier()

    def chunk_body(i, carry):
        pltpu.async_copy(x_hbm.at[src_v.at[i]], rows_v, sem).wait()
        pltpu.sync_copy(rows_v, acc.at[dst_v.at[i]], add=True)
        return carry

    lax.fori_loop(0, NCHUNK, chunk_body, 0)
    plsc.subcore_barrier()

    @pl.when(s == 0)
    def _copy_out():
        pltpu.sync_copy(acc, out_hbm.at[c])


_sc_feat = functools.partial(
    pl.kernel,
    mesh=_MESH,
    out_type=jax.ShapeDtypeStruct((NC, ACC_N, D), jnp.float32),
    scratch_types=[
        pltpu.VMEM((NCHUNK, CHUNK), jnp.int32),
        pltpu.VMEM((NCHUNK, CHUNK), jnp.int32),
        pltpu.VMEM((CHUNK, D), jnp.float32),
        pltpu.VMEM_SHARED((ACC_N, D), jnp.float32),
        pltpu.SemaphoreType.DMA,
    ],
)(_sc_feat_body)


def _sc_deg_body(dst_hbm, zdeg_hbm, ones_hbm, deg_out_hbm,
                 dst_v, ones_v, deg_acc):
    c = lax.axis_index("c")
    s = lax.axis_index("s")
    wid = s * NC + c

    @pl.when(s == 0)
    def _init():
        pltpu.sync_copy(zdeg_hbm, deg_acc)

    pltpu.sync_copy(dst_hbm.at[wid], dst_v)
    pltpu.sync_copy(ones_hbm, ones_v)
    plsc.subcore_barrier()

    def chunk_body(i, carry):
        pltpu.sync_copy(ones_v, deg_acc.at[dst_v.at[i]], add=True)
        return carry

    lax.fori_loop(0, NCHUNK, chunk_body, 0)
    plsc.subcore_barrier()

    @pl.when(s == 0)
    def _copy_out():
        pltpu.sync_copy(deg_acc, deg_out_hbm.at[c])


_sc_deg = functools.partial(
    pl.kernel,
    mesh=_MESH,
    out_type=jax.ShapeDtypeStruct((NC, ACC_N, DEGW), jnp.float32),
    scratch_types=[
        pltpu.VMEM((NCHUNK, CHUNK), jnp.int32),
        pltpu.VMEM((CHUNK, DEGW), jnp.float32),
        pltpu.VMEM_SHARED((ACC_N, DEGW), jnp.float32),
    ],
)(_sc_deg_body)


def _tc_layer_body(x_ref, p_ref, deg_ref, wl_ref, bl_ref, wr_ref, o_ref):
    agg_sum = p_ref[0, 0:N_NODES, :] + p_ref[1, 0:N_NODES, :]
    deg = deg_ref[0, 0:N_NODES, 0:1] + deg_ref[1, 0:N_NODES, 0:1]
    scale = 1.0 / jnp.maximum(deg, 1.0)
    agg = agg_sum * scale
    h = (jnp.dot(agg, wl_ref[...], preferred_element_type=jnp.float32)
         + bl_ref[...]
         + jnp.dot(x_ref[...], wr_ref[...], preferred_element_type=jnp.float32))
    o_ref[...] = jnp.tanh(h)


def _tc_proj_body(x_ref, wlin_ref, blin_ref, o_ref):
    o_ref[...] = (jnp.dot(x_ref[...], wlin_ref[...],
                          preferred_element_type=jnp.float32)
                  + blin_ref[...])


def _tc_layer(x, p, deg, wlT, bl, wrT):
    return pl.pallas_call(
        _tc_layer_body,
        out_shape=jax.ShapeDtypeStruct((N_NODES, D), jnp.float32),
    )(x, p, deg, wlT, bl, wrT)


def _tc_proj(x, wlinT, blin):
    return pl.pallas_call(
        _tc_proj_body,
        out_shape=jax.ShapeDtypeStruct((N_NODES, Wout), jnp.float32),
    )(x, wlinT, blin)


def kernel(x0, edge_index, W1l, b1l, W1r, W2l, b2l, W2r, W3l, b3l, W3r,
           Wlin, blin):
    pad = E_PAD - E
    src = jnp.concatenate(
        [edge_index[0].astype(jnp.int32), jnp.zeros((pad,), jnp.int32)]
    ).reshape(NW, NCHUNK, CHUNK)
    dst = jnp.concatenate(
        [edge_index[1].astype(jnp.int32),
         N_NODES + 16 + (jnp.arange(pad, dtype=jnp.int32) % 128)]
    ).reshape(NW, NCHUNK, CHUNK)
    zeros_acc = jnp.zeros((ACC_N, D), jnp.float32)
    zeros_deg = jnp.zeros((ACC_N, DEGW), jnp.float32)
    ones16 = jnp.ones((CHUNK, DEGW), jnp.float32)

    degp = _sc_feat(jnp.ones((N_NODES, D), jnp.float32), src, dst, zeros_acc)

    def sage(x, Wl, bl, Wr):
        p = _sc_feat(x, src, dst, zeros_acc)
        return _tc_layer(x, p, degp, Wl.T, bl.reshape(1, D), Wr.T)

    x1 = sage(x0, W1l, b1l, W1r)
    x2 = sage(x1, W2l, b2l, W2r)
    x3 = sage(x2, W3l, b3l, W3r)
    return _tc_proj(x3, Wlin.T, blin.reshape(1, Wout))


# speedup vs baseline: 3.4819x; 1.5153x over previous
import functools

import jax
import jax.numpy as jnp
from jax import lax
from jax.experimental import pallas as pl
from jax.experimental.pallas import tpu as pltpu
from jax.experimental.pallas import tpu_sc as plsc

N_NODES = 10000
D = 128
Wout = 64
E = 320000
NC = 2
NS = 16
NW = NC * NS
E_PAD = 327680
E_PER_TILE = E_PAD // NW
CHUNK = 128
NCHUNK = E_PER_TILE // CHUNK
NPAIR = NCHUNK // 2
ACC_N = 10240

_MESH = plsc.VectorSubcoreMesh(core_axis_name="c", subcore_axis_name="s")


def _sc_feat_body(x_hbm, src_hbm, dst_hbm, zeros_hbm, out_hbm,
                  src_v, dst_v, rows_a, acc, sem_a):
    c = lax.axis_index("c")
    s = lax.axis_index("s")
    wid = s * NC + c

    @pl.when(s == 0)
    def _init():
        pltpu.sync_copy(zeros_hbm, acc)

    pltpu.sync_copy(src_hbm.at[wid], src_v)
    pltpu.sync_copy(dst_hbm.at[wid], dst_v)
    plsc.subcore_barrier()

    def chunk_body(i, carry):
        pltpu.async_copy(x_hbm.at[src_v.at[i]], rows_a, sem_a).wait()
        pltpu.sync_copy(rows_a, acc.at[dst_v.at[i]], add=True)
        return carry

    lax.fori_loop(0, NCHUNK, chunk_body, 0)
    plsc.subcore_barrier()

    @pl.when(s == 0)
    def _copy_out():
        pltpu.sync_copy(acc, out_hbm.at[c])


_sc_feat = functools.partial(
    pl.kernel,
    mesh=_MESH,
    out_type=jax.ShapeDtypeStruct((NC, ACC_N, D), jnp.float32),
    scratch_types=[
        pltpu.VMEM((NCHUNK, CHUNK), jnp.int32),
        pltpu.VMEM((NCHUNK, CHUNK), jnp.int32),
        pltpu.VMEM((CHUNK, D), jnp.float32),
        pltpu.VMEM_SHARED((ACC_N, D), jnp.float32),
        pltpu.SemaphoreType.DMA,
    ],
)(_sc_feat_body)


def _sc_deg_body(dst_hbm, zeros_hbm, ones_hbm, out_hbm,
                 dst_v, ones_v, acc):
    c = lax.axis_index("c")
    s = lax.axis_index("s")
    wid = s * NC + c

    @pl.when(s == 0)
    def _init():
        pltpu.sync_copy(zeros_hbm, acc)

    pltpu.sync_copy(dst_hbm.at[wid], dst_v)
    pltpu.sync_copy(ones_hbm, ones_v)
    plsc.subcore_barrier()

    def chunk_body(i, carry):
        pltpu.sync_copy(ones_v, acc.at[dst_v.at[i]], add=True)
        return carry

    lax.fori_loop(0, NCHUNK, chunk_body, 0)
    plsc.subcore_barrier()

    @pl.when(s == 0)
    def _copy_out():
        pltpu.sync_copy(acc, out_hbm.at[c])


_sc_deg = functools.partial(
    pl.kernel,
    mesh=_MESH,
    out_type=jax.ShapeDtypeStruct((NC, ACC_N, D), jnp.float32),
    scratch_types=[
        pltpu.VMEM((NCHUNK, CHUNK), jnp.int32),
        pltpu.VMEM((CHUNK, D), jnp.float32),
        pltpu.VMEM_SHARED((ACC_N, D), jnp.float32),
    ],
)(_sc_deg_body)


def _tc_layer_body(x_ref, p_ref, deg_ref, wl_ref, bl_ref, wr_ref, o_ref):
    agg_sum = p_ref[0, 0:N_NODES, :] + p_ref[1, 0:N_NODES, :]
    deg = deg_ref[0, 0:N_NODES, 0:1] + deg_ref[1, 0:N_NODES, 0:1]
    scale = 1.0 / jnp.maximum(deg, 1.0)
    agg = agg_sum * scale
    h = (jnp.dot(agg, wl_ref[...], preferred_element_type=jnp.float32)
         + bl_ref[...]
         + jnp.dot(x_ref[...], wr_ref[...], preferred_element_type=jnp.float32))
    o_ref[...] = jnp.tanh(h)


def _tc_proj_body(x_ref, wlin_ref, blin_ref, o_ref):
    o_ref[...] = (jnp.dot(x_ref[...], wlin_ref[...],
                          preferred_element_type=jnp.float32)
                  + blin_ref[...])


def _tc_layer(x, p, deg, wlT, bl, wrT):
    return pl.pallas_call(
        _tc_layer_body,
        out_shape=jax.ShapeDtypeStruct((N_NODES, D), jnp.float32),
    )(x, p, deg, wlT, bl, wrT)


def _tc_proj(x, wlinT, blin):
    return pl.pallas_call(
        _tc_proj_body,
        out_shape=jax.ShapeDtypeStruct((N_NODES, Wout), jnp.float32),
    )(x, wlinT, blin)


def kernel(x0, edge_index, W1l, b1l, W1r, W2l, b2l, W2r, W3l, b3l, W3r,
           Wlin, blin):
    pad = E_PAD - E
    src = jnp.concatenate(
        [edge_index[0].astype(jnp.int32), jnp.zeros((pad,), jnp.int32)]
    ).reshape(NW, NCHUNK, CHUNK)
    dst = jnp.concatenate(
        [edge_index[1].astype(jnp.int32),
         N_NODES + 16 + (jnp.arange(pad, dtype=jnp.int32) % 128)]
    ).reshape(NW, NCHUNK, CHUNK)
    zeros_acc = jnp.zeros((ACC_N, D), jnp.float32)
    ones_wide = jnp.ones((CHUNK, D), jnp.float32)

    degp = _sc_deg(dst, zeros_acc, ones_wide)

    def sage(x, Wl, bl, Wr):
        p = _sc_feat(x, src, dst, zeros_acc)
        return _tc_layer(x, p, degp, Wl.T, bl.reshape(1, D), Wr.T)

    x1 = sage(x0, W1l, b1l, W1r)
    x2 = sage(x1, W2l, b2l, W2r)
    x3 = sage(x2, W3l, b3l, W3r)
    return _tc_proj(x3, Wlin.T, blin.reshape(1, Wout))


# speedup vs baseline: 3.6527x; 1.0491x over previous
import functools

import jax
import jax.numpy as jnp
from jax import lax
from jax.experimental import pallas as pl
from jax.experimental.pallas import tpu as pltpu
from jax.experimental.pallas import tpu_sc as plsc

N_NODES = 10000
D = 128
Wout = 64
E = 320000
NC = 2
NS = 16
NW = NC * NS
E_PAD = 327680
E_PER_TILE = E_PAD // NW
CHUNK = 128
NCHUNK = E_PER_TILE // CHUNK
BLK = 16
NBLK = NCHUNK // BLK
ACC_N = 10240

_MESH = plsc.VectorSubcoreMesh(core_axis_name="c", subcore_axis_name="s")


def _sc_feat_body(x_hbm, src_hbm, dst_hbm, zeros_hbm, out_hbm,
                  src_v, dst_v, rows_a, rows_b, acc, sem_a, sem_s):
    c = lax.axis_index("c")
    s = lax.axis_index("s")
    wid = s * NC + c

    @pl.when(s == 0)
    def _init():
        pltpu.sync_copy(zeros_hbm, acc)

    plsc.subcore_barrier()

    def gather(j, buf):
        pltpu.async_copy(x_hbm.at[src_v.at[j]], buf, sem_a).wait()

    def scat_start(buf, j):
        pltpu.async_copy(buf, acc.at[dst_v.at[j]], sem_s, add=True)

    def scat_wait(buf):
        pltpu.make_async_copy(zeros_hbm.at[pl.ds(0, CHUNK)], buf, sem_s).wait()

    def blk_body(b, carry):
        pltpu.sync_copy(src_hbm.at[wid].at[pl.ds(b * BLK, BLK)], src_v)
        pltpu.sync_copy(dst_hbm.at[wid].at[pl.ds(b * BLK, BLK)], dst_v)
        gather(0, rows_a)

        def pair_body(i, carry2):
            scat_start(rows_a, 2 * i)
            gather(2 * i + 1, rows_b)
            scat_wait(rows_a)
            scat_start(rows_b, 2 * i + 1)

            @pl.when(i < BLK // 2 - 1)
            def _next():
                gather(2 * i + 2, rows_a)

            scat_wait(rows_b)
            return carry2

        return lax.fori_loop(0, BLK // 2, pair_body, carry)

    lax.fori_loop(0, NBLK, blk_body, 0)
    plsc.subcore_barrier()

    @pl.when(s == 0)
    def _copy_out():
        pltpu.sync_copy(acc, out_hbm.at[c])


_sc_feat = functools.partial(
    pl.kernel,
    mesh=_MESH,
    out_type=jax.ShapeDtypeStruct((NC, ACC_N, D), jnp.float32),
    scratch_types=[
        pltpu.VMEM((BLK, CHUNK), jnp.int32),
        pltpu.VMEM((BLK, CHUNK), jnp.int32),
        pltpu.VMEM((CHUNK, D), jnp.float32),
        pltpu.VMEM((CHUNK, D), jnp.float32),
        pltpu.VMEM_SHARED((ACC_N, D), jnp.float32),
        pltpu.SemaphoreType.DMA,
        pltpu.SemaphoreType.DMA,
    ],
)(_sc_feat_body)


def _sc_deg_body(dst_hbm, zeros_hbm, ones_hbm, out_hbm,
                 dst_v, ones_v, acc):
    c = lax.axis_index("c")
    s = lax.axis_index("s")
    wid = s * NC + c

    @pl.when(s == 0)
    def _init():
        pltpu.sync_copy(zeros_hbm, acc)

    pltpu.sync_copy(dst_hbm.at[wid], dst_v)
    pltpu.sync_copy(ones_hbm, ones_v)
    plsc.subcore_barrier()

    def chunk_body(i, carry):
        pltpu.sync_copy(ones_v, acc.at[dst_v.at[i]], add=True)
        return carry

    lax.fori_loop(0, NCHUNK, chunk_body, 0)
    plsc.subcore_barrier()

    @pl.when(s == 0)
    def _copy_out():
        pltpu.sync_copy(acc, out_hbm.at[c])


_sc_deg = functools.partial(
    pl.kernel,
    mesh=_MESH,
    out_type=jax.ShapeDtypeStruct((NC, ACC_N, D), jnp.float32),
    scratch_types=[
        pltpu.VMEM((NCHUNK, CHUNK), jnp.int32),
        pltpu.VMEM((CHUNK, D), jnp.float32),
        pltpu.VMEM_SHARED((ACC_N, D), jnp.float32),
    ],
)(_sc_deg_body)


def _tc_layer_body(x_ref, p_ref, deg_ref, wl_ref, bl_ref, wr_ref, o_ref):
    agg_sum = p_ref[0, 0:N_NODES, :] + p_ref[1, 0:N_NODES, :]
    deg = deg_ref[0, 0:N_NODES, 0:1] + deg_ref[1, 0:N_NODES, 0:1]
    scale = 1.0 / jnp.maximum(deg, 1.0)
    agg = agg_sum * scale
    h = (jnp.dot(agg, wl_ref[...], preferred_element_type=jnp.float32)
         + bl_ref[...]
         + jnp.dot(x_ref[...], wr_ref[...], preferred_element_type=jnp.float32))
    o_ref[...] = jnp.tanh(h)


def _tc_proj_body(x_ref, wlin_ref, blin_ref, o_ref):
    o_ref[...] = (jnp.dot(x_ref[...], wlin_ref[...],
                          preferred_element_type=jnp.float32)
                  + blin_ref[...])


def _tc_layer(x, p, deg, wlT, bl, wrT):
    return pl.pallas_call(
        _tc_layer_body,
        out_shape=jax.ShapeDtypeStruct((N_NODES, D), jnp.float32),
    )(x, p, deg, wlT, bl, wrT)


def _tc_proj(x, wlinT, blin):
    return pl.pallas_call(
        _tc_proj_body,
        out_shape=jax.ShapeDtypeStruct((N_NODES, Wout), jnp.float32),
    )(x, wlinT, blin)


def kernel(x0, edge_index, W1l, b1l, W1r, W2l, b2l, W2r, W3l, b3l, W3r,
           Wlin, blin):
    pad = E_PAD - E
    src = jnp.concatenate(
        [edge_index[0].astype(jnp.int32), jnp.zeros((pad,), jnp.int32)]
    ).reshape(NW, NCHUNK, CHUNK)
    dst = jnp.concatenate(
        [edge_index[1].astype(jnp.int32),
         N_NODES + 16 + (jnp.arange(pad, dtype=jnp.int32) % 128)]
    ).reshape(NW, NCHUNK, CHUNK)
    zeros_acc = jnp.zeros((ACC_N, D), jnp.float32)
    ones_wide = jnp.ones((CHUNK, D), jnp.float32)

    degp = _sc_deg(dst, zeros_acc, ones_wide)

    def sage(x, Wl, bl, Wr):
        p = _sc_feat(x, src, dst, zeros_acc)
        return _tc_layer(x, p, degp, Wl.T, bl.reshape(1, D), Wr.T)

    x1 = sage(x0, W1l, b1l, W1r)
    x2 = sage(x1, W2l, b2l, W2r)
    x3 = sage(x2, W3l, b3l, W3r)
    return _tc_proj(x3, Wlin.T, blin.reshape(1, Wout))
